# baseline (speedup 1.0000x reference)
import functools

import jax
import jax.numpy as jnp
from jax import lax
from jax.experimental import pallas as pl
from jax.experimental.pallas import tpu as pltpu

_E = 320000
_EPAD = 327680
_EB = 2560
_NEB = _EPAD // _EB
_N = 10000
_NPAD = 10016
_NB = 2504
_OBJ_PAD = 136
_NPRED = 64
_H = 128


def _relu(x):
    return jnp.maximum(x, 0.0)


def _pre_body(obj_ref, pred_ref, ws_ref, wo_ref, wp_ref, b1_ref,
              ts_ref, to_ref, pt_ref):
    ts_ref[...] = obj_ref[...] @ ws_ref[...] + b1_ref[...]
    to_ref[...] = obj_ref[...] @ wo_ref[...]
    pt_ref[...] = pred_ref[...] @ wp_ref[...]


def _pre(obj_emb_p, pred_emb, ws, wo, wp, b1):
    return pl.pallas_call(
        _pre_body,
        out_shape=(jax.ShapeDtypeStruct((_OBJ_PAD, _H), jnp.float32),
                   jax.ShapeDtypeStruct((_OBJ_PAD, _H), jnp.float32),
                   jax.ShapeDtypeStruct((_NPRED, _H), jnp.float32)),
    )(obj_emb_p, pred_emb, ws, wo, wp, b1)


def _node0_body(objs_ref, ts_ref, to_ref, a_ref, b_ref):
    oh = (objs_ref[...] == lax.broadcasted_iota(jnp.int32, (_NB, _OBJ_PAD), 1)
          ).astype(jnp.float32)
    a_ref[...] = oh @ ts_ref[...]
    b_ref[...] = oh @ to_ref[...]


def _node0(objs2, ts, to):
    grid = (_NPAD // _NB,)
    return pl.pallas_call(
        _node0_body,
        grid=grid,
        in_specs=[pl.BlockSpec((_NB, 1), lambda i: (i, 0)),
                  pl.BlockSpec((_OBJ_PAD, _H), lambda i: (0, 0)),
                  pl.BlockSpec((_OBJ_PAD, _H), lambda i: (0, 0))],
        out_specs=(pl.BlockSpec((_NB, _H), lambda i: (i, 0)),
                   pl.BlockSpec((_NB, _H), lambda i: (i, 0))),
        out_shape=(jax.ShapeDtypeStruct((_NPAD, _H), jnp.float32),
                   jax.ShapeDtypeStruct((_NPAD, _H), jnp.float32)),
    )(objs2, ts, to)


def _big0_body(g_ref, p_ref, pt_ref, w2s_ref, w2p_ref, w2o_ref,
               b2s_ref, b2p_ref, b2o_ref, ns_ref, npv_ref, no_ref):
    oh = (p_ref[...] == lax.broadcasted_iota(jnp.int32, (_EB, _NPRED), 1)
          ).astype(jnp.float32)
    h = _relu(g_ref[...] + oh @ pt_ref[...])
    ns_ref[...] = _relu(h @ w2s_ref[...] + b2s_ref[...])
    npv_ref[...] = _relu(h @ w2p_ref[...] + b2p_ref[...])
    no_ref[...] = _relu(h @ w2o_ref[...] + b2o_ref[...])


def _big0(g, p2, pt, w2s, w2p, w2o, b2s, b2p, b2o):
    eb = pl.BlockSpec((_EB, _H), lambda i: (i, 0))
    full = lambda shape: pl.BlockSpec(shape, lambda i: (0, 0))
    return pl.pallas_call(
        _big0_body,
        grid=(_NEB,),
        in_specs=[eb,
                  pl.BlockSpec((_EB, 1), lambda i: (i, 0)),
                  full((_NPRED, _H)),
                  full((_H, _H)), full((_H, _H)), full((_H, _H)),
                  full((1, _H)), full((1, _H)), full((1, _H))],
        out_specs=(eb, eb, eb),
        out_shape=(jax.ShapeDtypeStruct((_EPAD, _H), jnp.float32),
                   jax.ShapeDtypeStruct((_EPAD, _H), jnp.float32),
                   jax.ShapeDtypeStruct((_EPAD, _H), jnp.float32)),
    )(g, p2, pt, w2s, w2p, w2o, b2s, b2p, b2o)


def _big_body(g_ref, pv_ref, wp_ref, w2s_ref, w2p_ref, w2o_ref,
              b2s_ref, b2p_ref, b2o_ref, ns_ref, npv_ref, no_ref):
    h = _relu(g_ref[...] + pv_ref[...] @ wp_ref[...])
    ns_ref[...] = _relu(h @ w2s_ref[...] + b2s_ref[...])
    npv_ref[...] = _relu(h @ w2p_ref[...] + b2p_ref[...])
    no_ref[...] = _relu(h @ w2o_ref[...] + b2o_ref[...])


def _big(g, pv, wp, w2s, w2p, w2o, b2s, b2p, b2o):
    eb = pl.BlockSpec((_EB, _H), lambda i: (i, 0))
    full = lambda shape: pl.BlockSpec(shape, lambda i: (0, 0))
    return pl.pallas_call(
        _big_body,
        grid=(_NEB,),
        in_specs=[eb, eb,
                  full((_H, _H)),
                  full((_H, _H)), full((_H, _H)), full((_H, _H)),
                  full((1, _H)), full((1, _H)), full((1, _H))],
        out_specs=(eb, eb, eb),
        out_shape=(jax.ShapeDtypeStruct((_EPAD, _H), jnp.float32),
                   jax.ShapeDtypeStruct((_EPAD, _H), jnp.float32),
                   jax.ShapeDtypeStruct((_EPAD, _H), jnp.float32)),
    )(g, pv, wp, w2s, w2p, w2o, b2s, b2p, b2o)


def _node_body(p0_ref, p1_ref, cnt_ref, w1_ref, nb1_ref, w2_ref, nb2_ref,
               wsn_ref, b1n_ref, won_ref, a_ref, b_ref):
    pooled = (p0_ref[...] + p1_ref[...]) / jnp.maximum(cnt_ref[...], 1.0)
    h2 = _relu(pooled @ w1_ref[...] + nb1_ref[...])
    ov = _relu(h2 @ w2_ref[...] + nb2_ref[...])
    a_ref[...] = ov @ wsn_ref[...] + b1n_ref[...]
    b_ref[...] = ov @ won_ref[...]


def _node(p0, p1, cnt, w1, nb1, w2, nb2, wsn, b1n, won):
    nb = pl.BlockSpec((_NB, _H), lambda i: (i, 0))
    full = lambda shape: pl.BlockSpec(shape, lambda i: (0, 0))
    return pl.pallas_call(
        _node_body,
        grid=(_NPAD // _NB,),
        in_specs=[nb, nb,
                  pl.BlockSpec((_NB, 1), lambda i: (i, 0)),
                  full((_H, _H)), full((1, _H)), full((_H, _H)), full((1, _H)),
                  full((_H, _H)), full((1, _H)), full((_H, _H))],
        out_specs=(nb, nb),
        out_shape=(jax.ShapeDtypeStruct((_NPAD, _H), jnp.float32),
                   jax.ShapeDtypeStruct((_NPAD, _H), jnp.float32)),
    )(p0, p1, cnt, w1, nb1, w2, nb2, wsn, b1n, won)


def _node_last_body(p0_ref, p1_ref, cnt_ref, w1_ref, nb1_ref, w2_ref, nb2_ref,
                    ov_ref):
    pooled = (p0_ref[...] + p1_ref[...]) / jnp.maximum(cnt_ref[...], 1.0)
    h2 = _relu(pooled @ w1_ref[...] + nb1_ref[...])
    ov_ref[...] = _relu(h2 @ w2_ref[...] + nb2_ref[...])


def _node_last(p0, p1, cnt, w1, nb1, w2, nb2):
    nb = pl.BlockSpec((_NB, _H), lambda i: (i, 0))
    full = lambda shape: pl.BlockSpec(shape, lambda i: (0, 0))
    return pl.pallas_call(
        _node_last_body,
        grid=(_NPAD // _NB,),
        in_specs=[nb, nb,
                  pl.BlockSpec((_NB, 1), lambda i: (i, 0)),
                  full((_H, _H)), full((1, _H)), full((_H, _H)), full((1, _H))],
        out_specs=nb,
        out_shape=jax.ShapeDtypeStruct((_NPAD, _H), jnp.float32),
    )(p0, p1, cnt, w1, nb1, w2, nb2)


def kernel(objs, triples, obj_emb, pred_emb,
           n1w1, n1b1, n1w2, n1b2, n2w1, n2b1, n2w2, n2b2):
    objs = objs.astype(jnp.int32)
    triples = triples.astype(jnp.int32)
    s = triples[:, 0]
    p = triples[:, 1]
    o = triples[:, 2]
    pad = _EPAD - _E
    padn = 10000 + (jnp.arange(pad, dtype=jnp.int32) % 16)
    s_p = jnp.concatenate([s, padn])
    o_p = jnp.concatenate([o, padn])
    p_p = jnp.concatenate([p, jnp.full((pad,), _NPRED, jnp.int32)])
    objs2 = jnp.concatenate(
        [objs, jnp.full((_NPAD - _N,), 129, jnp.int32)]).reshape(_NPAD, 1)
    obj_emb_p = jnp.pad(obj_emb, ((0, _OBJ_PAD - 129), (0, 0)))

    Ws = [n1w1[i, :_H] for i in range(4)]
    Wp = [n1w1[i, _H:2 * _H] for i in range(4)]
    Wo = [n1w1[i, 2 * _H:] for i in range(4)]
    b1 = [n1b1[i].reshape(1, _H) for i in range(4)]
    W2s = [n1w2[i][:, :_H] for i in range(4)]
    W2p = [n1w2[i][:, _H:2 * _H] for i in range(4)]
    W2o = [n1w2[i][:, 2 * _H:] for i in range(4)]
    b2s = [n1b2[i][:_H].reshape(1, _H) for i in range(4)]
    b2p = [n1b2[i][_H:2 * _H].reshape(1, _H) for i in range(4)]
    b2o = [n1b2[i][2 * _H:].reshape(1, _H) for i in range(4)]
    W1n = [n2w1[i] for i in range(4)]
    b1n_ = [n2b1[i].reshape(1, _H) for i in range(4)]
    W2n = [n2w2[i] for i in range(4)]
    b2n_ = [n2b2[i].reshape(1, _H) for i in range(4)]

    ts, to, pt = _pre(obj_emb_p, pred_emb, Ws[0], Wo[0], Wp[0], b1[0])
    A, B = _node0(objs2, ts, to)

    cnt = jnp.zeros((_NPAD,), jnp.float32).at[s_p].add(1.0).at[o_p].add(1.0)
    cnt2 = cnt.reshape(_NPAD, 1)
    zerosN = jnp.zeros((_NPAD, _H), jnp.float32)

    pv = None
    ov = None
    npv = None
    for i in range(4):
        g = jnp.take(A, s_p, axis=0) + jnp.take(B, o_p, axis=0)
        if i == 0:
            ns, npv, no = _big0(g, p_p.reshape(_EPAD, 1), pt,
                                W2s[i], W2p[i], W2o[i], b2s[i], b2p[i], b2o[i])
        else:
            ns, npv, no = _big(g, pv, Wp[i],
                               W2s[i], W2p[i], W2o[i], b2s[i], b2p[i], b2o[i])
        pooled = zerosN.at[s_p].add(ns).at[o_p].add(no)
        if i < 3:
            A, B = _node(pooled, zerosN, cnt2, W1n[i], b1n_[i], W2n[i], b2n_[i],
                         Ws[i + 1], b1[i + 1], Wo[i + 1])
        else:
            ov = _node_last(pooled, zerosN, cnt2, W1n[i], b1n_[i], W2n[i], b2n_[i])
        pv = npv
    return ov[:_N], npv[:_E]


# speedup vs baseline: 4.0113x; 4.0113x over previous
import functools

import jax
import jax.numpy as jnp
from jax import lax
from jax.experimental import pallas as pl
from jax.experimental.pallas import tpu as pltpu
from jax.experimental.pallas import tpu_sc as plsc

_E = 320000
_EPAD = 327680
_EB = 2560
_NEB = _EPAD // _EB
_N = 10000
_NPAD = 10112
_NB = 2528
_OBJ_PAD = 136
_NPRED = 64
_H = 128


def _relu(x):
    return jnp.maximum(x, 0.0)


def _pre_body(obj_ref, pred_ref, ws_ref, wo_ref, wp_ref, b1_ref,
              ts_ref, to_ref, pt_ref):
    ts_ref[...] = obj_ref[...] @ ws_ref[...] + b1_ref[...]
    to_ref[...] = obj_ref[...] @ wo_ref[...]
    pt_ref[...] = pred_ref[...] @ wp_ref[...]


def _pre(obj_emb_p, pred_emb, ws, wo, wp, b1):
    return pl.pallas_call(
        _pre_body,
        out_shape=(jax.ShapeDtypeStruct((_OBJ_PAD, _H), jnp.float32),
                   jax.ShapeDtypeStruct((_OBJ_PAD, _H), jnp.float32),
                   jax.ShapeDtypeStruct((_NPRED, _H), jnp.float32)),
    )(obj_emb_p, pred_emb, ws, wo, wp, b1)


def _node0_body(objs_ref, ts_ref, to_ref, a_ref, b_ref):
    oh = (objs_ref[...] == lax.broadcasted_iota(jnp.int32, (_NB, _OBJ_PAD), 1)
          ).astype(jnp.float32)
    a_ref[...] = oh @ ts_ref[...]
    b_ref[...] = oh @ to_ref[...]


def _node0(objs2, ts, to):
    grid = (_NPAD // _NB,)
    return pl.pallas_call(
        _node0_body,
        grid=grid,
        in_specs=[pl.BlockSpec((_NB, 1), lambda i: (i, 0)),
                  pl.BlockSpec((_OBJ_PAD, _H), lambda i: (0, 0)),
                  pl.BlockSpec((_OBJ_PAD, _H), lambda i: (0, 0))],
        out_specs=(pl.BlockSpec((_NB, _H), lambda i: (i, 0)),
                   pl.BlockSpec((_NB, _H), lambda i: (i, 0))),
        out_shape=(jax.ShapeDtypeStruct((_NPAD, _H), jnp.float32),
                   jax.ShapeDtypeStruct((_NPAD, _H), jnp.float32)),
    )(objs2, ts, to)


def _big0_body(g_ref, p_ref, pt_ref, w2s_ref, w2p_ref, w2o_ref,
               b2s_ref, b2p_ref, b2o_ref, ns_ref, npv_ref, no_ref):
    oh = (p_ref[...] == lax.broadcasted_iota(jnp.int32, (_EB, _NPRED), 1)
          ).astype(jnp.float32)
    h = _relu(g_ref[...] + oh @ pt_ref[...])
    ns_ref[...] = _relu(h @ w2s_ref[...] + b2s_ref[...])
    npv_ref[...] = _relu(h @ w2p_ref[...] + b2p_ref[...])
    no_ref[...] = _relu(h @ w2o_ref[...] + b2o_ref[...])


def _big0(g, p2, pt, w2s, w2p, w2o, b2s, b2p, b2o):
    eb = pl.BlockSpec((_EB, _H), lambda i: (i, 0))
    full = lambda shape: pl.BlockSpec(shape, lambda i: (0, 0))
    return pl.pallas_call(
        _big0_body,
        grid=(_NEB,),
        in_specs=[eb,
                  pl.BlockSpec((_EB, 1), lambda i: (i, 0)),
                  full((_NPRED, _H)),
                  full((_H, _H)), full((_H, _H)), full((_H, _H)),
                  full((1, _H)), full((1, _H)), full((1, _H))],
        out_specs=(eb, eb, eb),
        out_shape=(jax.ShapeDtypeStruct((_EPAD, _H), jnp.float32),
                   jax.ShapeDtypeStruct((_EPAD, _H), jnp.float32),
                   jax.ShapeDtypeStruct((_EPAD, _H), jnp.float32)),
    )(g, p2, pt, w2s, w2p, w2o, b2s, b2p, b2o)


def _big_body(g_ref, pv_ref, wp_ref, w2s_ref, w2p_ref, w2o_ref,
              b2s_ref, b2p_ref, b2o_ref, ns_ref, npv_ref, no_ref):
    h = _relu(g_ref[...] + pv_ref[...] @ wp_ref[...])
    ns_ref[...] = _relu(h @ w2s_ref[...] + b2s_ref[...])
    npv_ref[...] = _relu(h @ w2p_ref[...] + b2p_ref[...])
    no_ref[...] = _relu(h @ w2o_ref[...] + b2o_ref[...])


def _big(g, pv, wp, w2s, w2p, w2o, b2s, b2p, b2o):
    eb = pl.BlockSpec((_EB, _H), lambda i: (i, 0))
    full = lambda shape: pl.BlockSpec(shape, lambda i: (0, 0))
    return pl.pallas_call(
        _big_body,
        grid=(_NEB,),
        in_specs=[eb, eb,
                  full((_H, _H)),
                  full((_H, _H)), full((_H, _H)), full((_H, _H)),
                  full((1, _H)), full((1, _H)), full((1, _H))],
        out_specs=(eb, eb, eb),
        out_shape=(jax.ShapeDtypeStruct((_EPAD, _H), jnp.float32),
                   jax.ShapeDtypeStruct((_EPAD, _H), jnp.float32),
                   jax.ShapeDtypeStruct((_EPAD, _H), jnp.float32)),
    )(g, pv, wp, w2s, w2p, w2o, b2s, b2p, b2o)


def _node_body(p0_ref, p1_ref, cnt_ref, w1_ref, nb1_ref, w2_ref, nb2_ref,
               wsn_ref, b1n_ref, won_ref, a_ref, b_ref):
    pooled = (p0_ref[...] + p1_ref[...]) / jnp.maximum(cnt_ref[...], 1.0)
    h2 = _relu(pooled @ w1_ref[...] + nb1_ref[...])
    ov = _relu(h2 @ w2_ref[...] + nb2_ref[...])
    a_ref[...] = ov @ wsn_ref[...] + b1n_ref[...]
    b_ref[...] = ov @ won_ref[...]


def _node(p0, p1, cnt, w1, nb1, w2, nb2, wsn, b1n, won):
    nb = pl.BlockSpec((_NB, _H), lambda i: (i, 0))
    full = lambda shape: pl.BlockSpec(shape, lambda i: (0, 0))
    return pl.pallas_call(
        _node_body,
        grid=(_NPAD // _NB,),
        in_specs=[nb, nb,
                  pl.BlockSpec((_NB, 1), lambda i: (i, 0)),
                  full((_H, _H)), full((1, _H)), full((_H, _H)), full((1, _H)),
                  full((_H, _H)), full((1, _H)), full((_H, _H))],
        out_specs=(nb, nb),
        out_shape=(jax.ShapeDtypeStruct((_NPAD, _H), jnp.float32),
                   jax.ShapeDtypeStruct((_NPAD, _H), jnp.float32)),
    )(p0, p1, cnt, w1, nb1, w2, nb2, wsn, b1n, won)


def _node_last_body(p0_ref, p1_ref, cnt_ref, w1_ref, nb1_ref, w2_ref, nb2_ref,
                    ov_ref):
    pooled = (p0_ref[...] + p1_ref[...]) / jnp.maximum(cnt_ref[...], 1.0)
    h2 = _relu(pooled @ w1_ref[...] + nb1_ref[...])
    ov_ref[...] = _relu(h2 @ w2_ref[...] + nb2_ref[...])


def _node_last(p0, p1, cnt, w1, nb1, w2, nb2):
    nb = pl.BlockSpec((_NB, _H), lambda i: (i, 0))
    full = lambda shape: pl.BlockSpec(shape, lambda i: (0, 0))
    return pl.pallas_call(
        _node_last_body,
        grid=(_NPAD // _NB,),
        in_specs=[nb, nb,
                  pl.BlockSpec((_NB, 1), lambda i: (i, 0)),
                  full((_H, _H)), full((1, _H)), full((_H, _H)), full((1, _H))],
        out_specs=nb,
        out_shape=jax.ShapeDtypeStruct((_NPAD, _H), jnp.float32),
    )(p0, p1, cnt, w1, nb1, w2, nb2)


_NC = 2
_NS = 16
_NW = _NC * _NS
_NBLK = _EPAD // 128
_BPW = _NBLK // _NW
_BPC = _NBLK // _NC
_ROWS_PER_TILE = _NPAD // _NS

_sc_mesh = plsc.VectorSubcoreMesh(
    core_axis_name="c", subcore_axis_name="s", num_cores=_NC, num_subcores=_NS)


@functools.partial(
    pl.kernel,
    out_type=jax.ShapeDtypeStruct((_EPAD, _H), jnp.float32),
    mesh=_sc_mesh,
    scratch_types=[pltpu.VMEM((2, 128), jnp.int32),
                   pltpu.VMEM((128, _H), jnp.float32),
                   pltpu.SemaphoreType.DMA],
)
def _gather_add(a_hbm, b_hbm, idx_hbm, g_hbm, idx_v, buf_v, sem):
    c = lax.axis_index("c")
    t = lax.axis_index("s")
    wid = c * _NS + t
    base = wid * _BPW

    def body(j, carry):
        blk = base + j
        pltpu.sync_copy(idx_hbm.at[blk], idx_v)
        pltpu.async_copy(a_hbm.at[idx_v.at[0]], buf_v, sem).wait()
        pltpu.async_copy(b_hbm.at[idx_v.at[1]], buf_v, sem, add=True).wait()
        pltpu.sync_copy(buf_v, g_hbm.at[pl.ds(blk * 128, 128)])
        return carry

    lax.fori_loop(0, _BPW, body, 0)


@functools.partial(
    pl.kernel,
    out_type=jax.ShapeDtypeStruct((_NC, _NPAD, _H), jnp.float32),
    mesh=_sc_mesh,
    scratch_types=[pltpu.VMEM((2, 128), jnp.int32),
                   pltpu.VMEM((128, _H), jnp.float32),
                   pltpu.VMEM_SHARED((_NPAD, _H), jnp.float32),
                   pltpu.SemaphoreType.DMA],
)
def _scatter2(ns_hbm, no_hbm, idx_hbm, zrows_hbm, out_hbm,
              idx_v, buf_v, acc_sh, sem):
    c = lax.axis_index("c")
    t = lax.axis_index("s")
    rbase = t * _ROWS_PER_TILE
    pltpu.sync_copy(zrows_hbm, acc_sh.at[pl.ds(rbase, _ROWS_PER_TILE)])
    plsc.subcore_barrier()
    base = (c * _NS + t) * _BPW

    def body(j, carry):
        blk = base + j
        pltpu.sync_copy(idx_hbm.at[blk], idx_v)
        pltpu.async_copy(ns_hbm.at[pl.ds(blk * 128, 128)], buf_v, sem).wait()
        pltpu.sync_copy(buf_v, acc_sh.at[idx_v.at[0]], add=True)
        pltpu.async_copy(no_hbm.at[pl.ds(blk * 128, 128)], buf_v, sem).wait()
        pltpu.sync_copy(buf_v, acc_sh.at[idx_v.at[1]], add=True)
        return carry

    lax.fori_loop(0, _BPW, body, 0)
    plsc.subcore_barrier()
    pltpu.sync_copy(acc_sh.at[pl.ds(rbase, _ROWS_PER_TILE)],
                    out_hbm.at[c].at[pl.ds(rbase, _ROWS_PER_TILE)])


_CNT_RPT = 640
_CNT_PAD = _CNT_RPT * _NS


@functools.partial(
    pl.kernel,
    out_type=jax.ShapeDtypeStruct((_NC, _CNT_PAD), jnp.float32),
    mesh=_sc_mesh,
    scratch_types=[pltpu.VMEM((2, 128), jnp.int32),
                   pltpu.VMEM((128,), jnp.float32),
                   pltpu.VMEM_SHARED((_CNT_PAD,), jnp.float32),
                   pltpu.SemaphoreType.DMA],
)
def _counts(idx_hbm, ones_hbm, z1_hbm, out_hbm, idx_v, ones_v, cnt_sh, sem):
    c = lax.axis_index("c")
    t = lax.axis_index("s")
    rbase = t * _CNT_RPT
    pltpu.sync_copy(z1_hbm, cnt_sh.at[pl.ds(rbase, _CNT_RPT)])
    pltpu.sync_copy(ones_hbm, ones_v)
    plsc.subcore_barrier()
    base = (c * _NS + t) * _BPW

    def body(j, carry):
        blk = base + j
        pltpu.sync_copy(idx_hbm.at[blk], idx_v)
        pltpu.sync_copy(ones_v, cnt_sh.at[idx_v.at[0]], add=True)
        pltpu.sync_copy(ones_v, cnt_sh.at[idx_v.at[1]], add=True)
        return carry

    lax.fori_loop(0, _BPW, body, 0)
    plsc.subcore_barrier()
    pltpu.sync_copy(cnt_sh.at[pl.ds(rbase, _CNT_RPT)],
                    out_hbm.at[c].at[pl.ds(rbase, _CNT_RPT)])


def kernel(objs, triples, obj_emb, pred_emb,
           n1w1, n1b1, n1w2, n1b2, n2w1, n2b1, n2w2, n2b2):
    objs = objs.astype(jnp.int32)
    triples = triples.astype(jnp.int32)
    s = triples[:, 0]
    p = triples[:, 1]
    o = triples[:, 2]
    pad = _EPAD - _E
    padn = 10000 + (jnp.arange(pad, dtype=jnp.int32) % 16)
    s_p = jnp.concatenate([s, padn])
    o_p = jnp.concatenate([o, padn])
    p_p = jnp.concatenate([p, jnp.full((pad,), _NPRED, jnp.int32)])
    objs2 = jnp.concatenate(
        [objs, jnp.full((_NPAD - _N,), 129, jnp.int32)]).reshape(_NPAD, 1)
    obj_emb_p = jnp.pad(obj_emb, ((0, _OBJ_PAD - 129), (0, 0)))

    Ws = [n1w1[i, :_H] for i in range(4)]
    Wp = [n1w1[i, _H:2 * _H] for i in range(4)]
    Wo = [n1w1[i, 2 * _H:] for i in range(4)]
    b1 = [n1b1[i].reshape(1, _H) for i in range(4)]
    W2s = [n1w2[i][:, :_H] for i in range(4)]
    W2p = [n1w2[i][:, _H:2 * _H] for i in range(4)]
    W2o = [n1w2[i][:, 2 * _H:] for i in range(4)]
    b2s = [n1b2[i][:_H].reshape(1, _H) for i in range(4)]
    b2p = [n1b2[i][_H:2 * _H].reshape(1, _H) for i in range(4)]
    b2o = [n1b2[i][2 * _H:].reshape(1, _H) for i in range(4)]
    W1n = [n2w1[i] for i in range(4)]
    b1n_ = [n2b1[i].reshape(1, _H) for i in range(4)]
    W2n = [n2w2[i] for i in range(4)]
    b2n_ = [n2b2[i].reshape(1, _H) for i in range(4)]

    so_idx = jnp.stack([s_p.reshape(_NBLK, 128), o_p.reshape(_NBLK, 128)],
                       axis=1)

    ts, to, pt = _pre(obj_emb_p, pred_emb, Ws[0], Wo[0], Wp[0], b1[0])
    A, B = _node0(objs2, ts, to)

    ones128 = jnp.ones((128,), jnp.float32)
    z1 = jnp.zeros((_CNT_RPT,), jnp.float32)
    zrows = jnp.zeros((_ROWS_PER_TILE, _H), jnp.float32)
    cnt_part = _counts(so_idx, ones128, z1)
    cnt2 = (cnt_part[0, :_NPAD] + cnt_part[1, :_NPAD]).reshape(_NPAD, 1)

    pv = None
    ov = None
    npv = None
    for i in range(4):
        g = _gather_add(A, B, so_idx)
        if i == 0:
            ns, npv, no = _big0(g, p_p.reshape(_EPAD, 1), pt,
                                W2s[i], W2p[i], W2o[i], b2s[i], b2p[i], b2o[i])
        else:
            ns, npv, no = _big(g, pv, Wp[i],
                               W2s[i], W2p[i], W2o[i], b2s[i], b2p[i], b2o[i])
        pooled = _scatter2(ns, no, so_idx, zrows)
        if i < 3:
            A, B = _node(pooled[0], pooled[1], cnt2,
                         W1n[i], b1n_[i], W2n[i], b2n_[i],
                         Ws[i + 1], b1[i + 1], Wo[i + 1])
        else:
            ov = _node_last(pooled[0], pooled[1], cnt2,
                            W1n[i], b1n_[i], W2n[i], b2n_[i])
        pv = npv
    return ov[:_N], npv[:_E]


# speedup vs baseline: 4.6927x; 1.1699x over previous
import functools

import jax
import jax.numpy as jnp
from jax import lax
from jax.experimental import pallas as pl
from jax.experimental.pallas import tpu as pltpu
from jax.experimental.pallas import tpu_sc as plsc

_E = 320000
_EPAD = 327680
_EB = 2560
_NEB = _EPAD // _EB
_N = 10000
_NPAD = 10112
_NB = 2528
_OBJ_PAD = 136
_NPRED = 64
_H = 128


def _relu(x):
    return jnp.maximum(x, 0.0)


def _pre_body(obj_ref, pred_ref, ws_ref, wo_ref, wp_ref, b1_ref,
              ts_ref, to_ref, pt_ref):
    ts_ref[...] = obj_ref[...] @ ws_ref[...] + b1_ref[...]
    to_ref[...] = obj_ref[...] @ wo_ref[...]
    pt_ref[...] = pred_ref[...] @ wp_ref[...]


def _pre(obj_emb_p, pred_emb, ws, wo, wp, b1):
    return pl.pallas_call(
        _pre_body,
        out_shape=(jax.ShapeDtypeStruct((_OBJ_PAD, _H), jnp.float32),
                   jax.ShapeDtypeStruct((_OBJ_PAD, _H), jnp.float32),
                   jax.ShapeDtypeStruct((_NPRED, _H), jnp.float32)),
    )(obj_emb_p, pred_emb, ws, wo, wp, b1)


def _node0_body(objs_ref, ts_ref, to_ref, a_ref, b_ref):
    oh = (objs_ref[...] == lax.broadcasted_iota(jnp.int32, (_NB, _OBJ_PAD), 1)
          ).astype(jnp.float32)
    a_ref[...] = oh @ ts_ref[...]
    b_ref[...] = oh @ to_ref[...]


def _node0(objs2, ts, to):
    grid = (_NPAD // _NB,)
    return pl.pallas_call(
        _node0_body,
        grid=grid,
        in_specs=[pl.BlockSpec((_NB, 1), lambda i: (i, 0)),
                  pl.BlockSpec((_OBJ_PAD, _H), lambda i: (0, 0)),
                  pl.BlockSpec((_OBJ_PAD, _H), lambda i: (0, 0))],
        out_specs=(pl.BlockSpec((_NB, _H), lambda i: (i, 0)),
                   pl.BlockSpec((_NB, _H), lambda i: (i, 0))),
        out_shape=(jax.ShapeDtypeStruct((_NPAD, _H), jnp.float32),
                   jax.ShapeDtypeStruct((_NPAD, _H), jnp.float32)),
    )(objs2, ts, to)


def _big0_body(g_ref, p_ref, pt_ref, w2s_ref, w2p_ref, w2o_ref,
               b2s_ref, b2p_ref, b2o_ref, ns_ref, npv_ref, no_ref):
    oh = (p_ref[...] == lax.broadcasted_iota(jnp.int32, (_EB, _NPRED), 1)
          ).astype(jnp.float32)
    h = _relu(g_ref[...] + oh @ pt_ref[...])
    ns_ref[...] = _relu(h @ w2s_ref[...] + b2s_ref[...])
    npv_ref[...] = _relu(h @ w2p_ref[...] + b2p_ref[...])
    no_ref[...] = _relu(h @ w2o_ref[...] + b2o_ref[...])


def _big0(g, p2, pt, w2s, w2p, w2o, b2s, b2p, b2o):
    eb = pl.BlockSpec((_EB, _H), lambda i: (i, 0))
    full = lambda shape: pl.BlockSpec(shape, lambda i: (0, 0))
    return pl.pallas_call(
        _big0_body,
        grid=(_NEB,),
        in_specs=[eb,
                  pl.BlockSpec((_EB, 1), lambda i: (i, 0)),
                  full((_NPRED, _H)),
                  full((_H, _H)), full((_H, _H)), full((_H, _H)),
                  full((1, _H)), full((1, _H)), full((1, _H))],
        out_specs=(eb, eb, eb),
        out_shape=(jax.ShapeDtypeStruct((_EPAD, _H), jnp.float32),
                   jax.ShapeDtypeStruct((_EPAD, _H), jnp.float32),
                   jax.ShapeDtypeStruct((_EPAD, _H), jnp.float32)),
    )(g, p2, pt, w2s, w2p, w2o, b2s, b2p, b2o)


def _big_body(g_ref, pv_ref, wp_ref, w2s_ref, w2p_ref, w2o_ref,
              b2s_ref, b2p_ref, b2o_ref, ns_ref, npv_ref, no_ref):
    h = _relu(g_ref[...] + pv_ref[...] @ wp_ref[...])
    ns_ref[...] = _relu(h @ w2s_ref[...] + b2s_ref[...])
    npv_ref[...] = _relu(h @ w2p_ref[...] + b2p_ref[...])
    no_ref[...] = _relu(h @ w2o_ref[...] + b2o_ref[...])


def _big(g, pv, wp, w2s, w2p, w2o, b2s, b2p, b2o):
    eb = pl.BlockSpec((_EB, _H), lambda i: (i, 0))
    full = lambda shape: pl.BlockSpec(shape, lambda i: (0, 0))
    return pl.pallas_call(
        _big_body,
        grid=(_NEB,),
        in_specs=[eb, eb,
                  full((_H, _H)),
                  full((_H, _H)), full((_H, _H)), full((_H, _H)),
                  full((1, _H)), full((1, _H)), full((1, _H))],
        out_specs=(eb, eb, eb),
        out_shape=(jax.ShapeDtypeStruct((_EPAD, _H), jnp.float32),
                   jax.ShapeDtypeStruct((_EPAD, _H), jnp.float32),
                   jax.ShapeDtypeStruct((_EPAD, _H), jnp.float32)),
    )(g, pv, wp, w2s, w2p, w2o, b2s, b2p, b2o)


def _node_body(p0_ref, p1_ref, cnt_ref, w1_ref, nb1_ref, w2_ref, nb2_ref,
               wsn_ref, b1n_ref, won_ref, a_ref, b_ref):
    pooled = (p0_ref[...] + p1_ref[...]) / jnp.maximum(cnt_ref[...], 1.0)
    h2 = _relu(pooled @ w1_ref[...] + nb1_ref[...])
    ov = _relu(h2 @ w2_ref[...] + nb2_ref[...])
    a_ref[...] = ov @ wsn_ref[...] + b1n_ref[...]
    b_ref[...] = ov @ won_ref[...]


def _node(p0, p1, cnt, w1, nb1, w2, nb2, wsn, b1n, won):
    nb = pl.BlockSpec((_NB, _H), lambda i: (i, 0))
    full = lambda shape: pl.BlockSpec(shape, lambda i: (0, 0))
    return pl.pallas_call(
        _node_body,
        grid=(_NPAD // _NB,),
        in_specs=[nb, nb,
                  pl.BlockSpec((_NB, 1), lambda i: (i, 0)),
                  full((_H, _H)), full((1, _H)), full((_H, _H)), full((1, _H)),
                  full((_H, _H)), full((1, _H)), full((_H, _H))],
        out_specs=(nb, nb),
        out_shape=(jax.ShapeDtypeStruct((_NPAD, _H), jnp.float32),
                   jax.ShapeDtypeStruct((_NPAD, _H), jnp.float32)),
    )(p0, p1, cnt, w1, nb1, w2, nb2, wsn, b1n, won)


def _node_last_body(p0_ref, p1_ref, cnt_ref, w1_ref, nb1_ref, w2_ref, nb2_ref,
                    ov_ref):
    pooled = (p0_ref[...] + p1_ref[...]) / jnp.maximum(cnt_ref[...], 1.0)
    h2 = _relu(pooled @ w1_ref[...] + nb1_ref[...])
    ov_ref[...] = _relu(h2 @ w2_ref[...] + nb2_ref[...])


def _node_last(p0, p1, cnt, w1, nb1, w2, nb2):
    nb = pl.BlockSpec((_NB, _H), lambda i: (i, 0))
    full = lambda shape: pl.BlockSpec(shape, lambda i: (0, 0))
    return pl.pallas_call(
        _node_last_body,
        grid=(_NPAD // _NB,),
        in_specs=[nb, nb,
                  pl.BlockSpec((_NB, 1), lambda i: (i, 0)),
                  full((_H, _H)), full((1, _H)), full((_H, _H)), full((1, _H))],
        out_specs=nb,
        out_shape=jax.ShapeDtypeStruct((_NPAD, _H), jnp.float32),
    )(p0, p1, cnt, w1, nb1, w2, nb2)


_NC = 2
_NS = 16
_NW = _NC * _NS
_SB = 512
_NBLK = _EPAD // _SB
_BPW = _NBLK // _NW
_EPW = _EPAD // _NW
_SSB = 256
_SBPW = _EPAD // _SSB // _NW
_ROWS_PER_TILE = _NPAD // _NS

_sc_mesh = plsc.VectorSubcoreMesh(
    core_axis_name="c", subcore_axis_name="s", num_cores=_NC, num_subcores=_NS)


@functools.partial(
    pl.kernel,
    out_type=jax.ShapeDtypeStruct((_EPAD, _H), jnp.float32),
    mesh=_sc_mesh,
    scratch_types=[pltpu.VMEM((2 * _EPW,), jnp.int32),
                   pltpu.VMEM((_SB, _H), jnp.float32),
                   pltpu.SemaphoreType.DMA],
)
def _gather_add(a_hbm, b_hbm, idx_hbm, g_hbm, idx_v, buf_v, sem):
    c = lax.axis_index("c")
    t = lax.axis_index("s")
    wid = c * _NS + t
    base = wid * _BPW
    pltpu.sync_copy(idx_hbm.at[wid], idx_v)

    def body(j, carry):
        blk = base + j
        off = pl.multiple_of(j * (2 * _SB), 2 * _SB)
        pltpu.async_copy(a_hbm.at[idx_v.at[pl.ds(off, _SB)]],
                         buf_v, sem).wait()
        pltpu.async_copy(b_hbm.at[idx_v.at[pl.ds(off + _SB, _SB)]],
                         buf_v, sem, add=True).wait()
        pltpu.sync_copy(buf_v, g_hbm.at[pl.ds(blk * _SB, _SB)])
        return carry

    lax.fori_loop(0, _BPW, body, 0)


@functools.partial(
    pl.kernel,
    out_type=jax.ShapeDtypeStruct((_NC, _NPAD, _H), jnp.float32),
    mesh=_sc_mesh,
    scratch_types=[pltpu.VMEM((_SSB,), jnp.int32),
                   pltpu.VMEM((_SSB,), jnp.int32),
                   pltpu.VMEM((_SSB, _H), jnp.float32),
                   pltpu.VMEM_SHARED((_NPAD, _H), jnp.float32),
                   pltpu.SemaphoreType.DMA],
)
def _scatter2(ns_hbm, no_hbm, idx_hbm, zrows_hbm, out_hbm,
              idx_s, idx_o, buf_v, acc_sh, sem):
    c = lax.axis_index("c")
    t = lax.axis_index("s")
    rbase = t * _ROWS_PER_TILE
    pltpu.sync_copy(zrows_hbm, acc_sh.at[pl.ds(rbase, _ROWS_PER_TILE)])
    base = (c * _NS + t) * _SBPW
    plsc.subcore_barrier()

    def body(j, carry):
        blk = base + j
        pltpu.sync_copy(idx_hbm.at[2 * blk], idx_s)
        pltpu.async_copy(ns_hbm.at[pl.ds(blk * _SSB, _SSB)], buf_v, sem).wait()
        pltpu.sync_copy(buf_v, acc_sh.at[idx_s], add=True)
        pltpu.sync_copy(idx_hbm.at[2 * blk + 1], idx_o)
        pltpu.async_copy(no_hbm.at[pl.ds(blk * _SSB, _SSB)], buf_v, sem).wait()
        pltpu.sync_copy(buf_v, acc_sh.at[idx_o], add=True)
        return carry

    lax.fori_loop(0, _SBPW, body, 0)
    plsc.subcore_barrier()
    pltpu.sync_copy(acc_sh.at[pl.ds(rbase, _ROWS_PER_TILE)],
                    out_hbm.at[c].at[pl.ds(rbase, _ROWS_PER_TILE)])


_CNT_RPT = 640
_CNT_PAD = _CNT_RPT * _NS


@functools.partial(
    pl.kernel,
    out_type=jax.ShapeDtypeStruct((_NC, _CNT_PAD), jnp.float32),
    mesh=_sc_mesh,
    scratch_types=[pltpu.VMEM((2 * _EPW,), jnp.int32),
                   pltpu.VMEM((2 * _EPW,), jnp.float32),
                   pltpu.VMEM_SHARED((_CNT_PAD,), jnp.float32),
                   pltpu.SemaphoreType.DMA],
)
def _counts(cidx_hbm, ones_hbm, z1_hbm, out_hbm, cidx_v, ones_v, cnt_sh, sem):
    c = lax.axis_index("c")
    t = lax.axis_index("s")
    wid = c * _NS + t
    rbase = t * _CNT_RPT
    pltpu.sync_copy(z1_hbm, cnt_sh.at[pl.ds(rbase, _CNT_RPT)])
    pltpu.sync_copy(cidx_hbm.at[wid], cidx_v)
    pltpu.sync_copy(ones_hbm, ones_v)
    plsc.subcore_barrier()
    pltpu.sync_copy(ones_v, cnt_sh.at[cidx_v], add=True)
    plsc.subcore_barrier()
    pltpu.sync_copy(cnt_sh.at[pl.ds(rbase, _CNT_RPT)],
                    out_hbm.at[c].at[pl.ds(rbase, _CNT_RPT)])


def kernel(objs, triples, obj_emb, pred_emb,
           n1w1, n1b1, n1w2, n1b2, n2w1, n2b1, n2w2, n2b2):
    objs = objs.astype(jnp.int32)
    triples = triples.astype(jnp.int32)
    s = triples[:, 0]
    p = triples[:, 1]
    o = triples[:, 2]
    pad = _EPAD - _E
    padn = 10000 + (jnp.arange(pad, dtype=jnp.int32) % 16)
    s_p = jnp.concatenate([s, padn])
    o_p = jnp.concatenate([o, padn])
    p_p = jnp.concatenate([p, jnp.full((pad,), _NPRED, jnp.int32)])
    objs2 = jnp.concatenate(
        [objs, jnp.full((_NPAD - _N,), 129, jnp.int32)]).reshape(_NPAD, 1)
    obj_emb_p = jnp.pad(obj_emb, ((0, _OBJ_PAD - 129), (0, 0)))

    Ws = [n1w1[i, :_H] for i in range(4)]
    Wp = [n1w1[i, _H:2 * _H] for i in range(4)]
    Wo = [n1w1[i, 2 * _H:] for i in range(4)]
    b1 = [n1b1[i].reshape(1, _H) for i in range(4)]
    W2s = [n1w2[i][:, :_H] for i in range(4)]
    W2p = [n1w2[i][:, _H:2 * _H] for i in range(4)]
    W2o = [n1w2[i][:, 2 * _H:] for i in range(4)]
    b2s = [n1b2[i][:_H].reshape(1, _H) for i in range(4)]
    b2p = [n1b2[i][_H:2 * _H].reshape(1, _H) for i in range(4)]
    b2o = [n1b2[i][2 * _H:].reshape(1, _H) for i in range(4)]
    W1n = [n2w1[i] for i in range(4)]
    b1n_ = [n2b1[i].reshape(1, _H) for i in range(4)]
    W2n = [n2w2[i] for i in range(4)]
    b2n_ = [n2b2[i].reshape(1, _H) for i in range(4)]

    so_idx = jnp.stack([s_p.reshape(_NBLK, _SB), o_p.reshape(_NBLK, _SB)],
                       axis=1).reshape(2 * _NBLK, _SB)
    so_idx_w = so_idx.reshape(_NW, 2 * _EPW)
    scat_idx = jnp.stack(
        [s_p.reshape(_EPAD // _SSB, _SSB), o_p.reshape(_EPAD // _SSB, _SSB)],
        axis=1).reshape(2 * _EPAD // _SSB, _SSB)

    ts, to, pt = _pre(obj_emb_p, pred_emb, Ws[0], Wo[0], Wp[0], b1[0])
    A, B = _node0(objs2, ts, to)

    ones_w = jnp.ones((2 * _EPW,), jnp.float32)
    z1 = jnp.zeros((_CNT_RPT,), jnp.float32)
    zrows = jnp.zeros((_ROWS_PER_TILE, _H), jnp.float32)
    cnt_part = _counts(so_idx_w, ones_w, z1)
    cnt2 = (cnt_part[0, :_NPAD] + cnt_part[1, :_NPAD]).reshape(_NPAD, 1)

    pv = None
    ov = None
    npv = None
    for i in range(4):
        g = _gather_add(A, B, so_idx_w)
        if i == 0:
            ns, npv, no = _big0(g, p_p.reshape(_EPAD, 1), pt,
                                W2s[i], W2p[i], W2o[i], b2s[i], b2p[i], b2o[i])
        else:
            ns, npv, no = _big(g, pv, Wp[i],
                               W2s[i], W2p[i], W2o[i], b2s[i], b2p[i], b2o[i])
        pooled = _scatter2(ns, no, scat_idx, zrows)
        if i < 3:
            A, B = _node(pooled[0], pooled[1], cnt2,
                         W1n[i], b1n_[i], W2n[i], b2n_[i],
                         Ws[i + 1], b1[i + 1], Wo[i + 1])
        else:
            ov = _node_last(pooled[0], pooled[1], cnt2,
                            W1n[i], b1n_[i], W2n[i], b2n_[i])
        pv = npv
    return ov[:_N], npv[:_E]


# speedup vs baseline: 4.9314x; 1.0509x over previous
import functools

import jax
import jax.numpy as jnp
from jax import lax
from jax.experimental import pallas as pl
from jax.experimental.pallas import tpu as pltpu
from jax.experimental.pallas import tpu_sc as plsc

_E = 320000
_EPAD = 327680
_EB = 2560
_NEB = _EPAD // _EB
_N = 10000
_NPAD = 10112
_NB = 2528
_OBJ_PAD = 136
_NPRED = 64
_H = 128


def _relu(x):
    return jnp.maximum(x, 0.0)


def _pre_body(obj_ref, pred_ref, ws_ref, wo_ref, wp_ref, b1_ref,
              ts_ref, to_ref, pt_ref):
    ts_ref[...] = obj_ref[...] @ ws_ref[...] + b1_ref[...]
    to_ref[...] = obj_ref[...] @ wo_ref[...]
    pt_ref[...] = pred_ref[...] @ wp_ref[...]


def _pre(obj_emb_p, pred_emb, ws, wo, wp, b1):
    return pl.pallas_call(
        _pre_body,
        out_shape=(jax.ShapeDtypeStruct((_OBJ_PAD, _H), jnp.float32),
                   jax.ShapeDtypeStruct((_OBJ_PAD, _H), jnp.float32),
                   jax.ShapeDtypeStruct((_NPRED, _H), jnp.float32)),
    )(obj_emb_p, pred_emb, ws, wo, wp, b1)


def _node0_body(objs_ref, ts_ref, to_ref, a_ref, b_ref):
    oh = (objs_ref[...] == lax.broadcasted_iota(jnp.int32, (_NB, _OBJ_PAD), 1)
          ).astype(jnp.float32)
    a_ref[...] = oh @ ts_ref[...]
    b_ref[...] = oh @ to_ref[...]


def _node0(objs2, ts, to):
    grid = (_NPAD // _NB,)
    return pl.pallas_call(
        _node0_body,
        grid=grid,
        in_specs=[pl.BlockSpec((_NB, 1), lambda i: (i, 0)),
                  pl.BlockSpec((_OBJ_PAD, _H), lambda i: (0, 0)),
                  pl.BlockSpec((_OBJ_PAD, _H), lambda i: (0, 0))],
        out_specs=(pl.BlockSpec((_NB, _H), lambda i: (i, 0)),
                   pl.BlockSpec((_NB, _H), lambda i: (i, 0))),
        out_shape=(jax.ShapeDtypeStruct((_NPAD, _H), jnp.float32),
                   jax.ShapeDtypeStruct((_NPAD, _H), jnp.float32)),
    )(objs2, ts, to)


def _big0_body(g_ref, p_ref, pt_ref, w2s_ref, w2p_ref, w2o_ref,
               b2s_ref, b2p_ref, b2o_ref, ns_ref, npv_ref, no_ref):
    oh = (p_ref[...] == lax.broadcasted_iota(jnp.int32, (_EB, _NPRED), 1)
          ).astype(jnp.float32)
    h = _relu(g_ref[...] + oh @ pt_ref[...])
    ns_ref[...] = _relu(h @ w2s_ref[...] + b2s_ref[...])
    npv_ref[...] = _relu(h @ w2p_ref[...] + b2p_ref[...])
    no_ref[...] = _relu(h @ w2o_ref[...] + b2o_ref[...])


def _big0(g, p2, pt, w2s, w2p, w2o, b2s, b2p, b2o):
    eb = pl.BlockSpec((_EB, _H), lambda i: (i, 0))
    full = lambda shape: pl.BlockSpec(shape, lambda i: (0, 0))
    return pl.pallas_call(
        _big0_body,
        grid=(_NEB,),
        in_specs=[eb,
                  pl.BlockSpec((_EB, 1), lambda i: (i, 0)),
                  full((_NPRED, _H)),
                  full((_H, _H)), full((_H, _H)), full((_H, _H)),
                  full((1, _H)), full((1, _H)), full((1, _H))],
        out_specs=(eb, eb, eb),
        out_shape=(jax.ShapeDtypeStruct((_EPAD, _H), jnp.float32),
                   jax.ShapeDtypeStruct((_EPAD, _H), jnp.float32),
                   jax.ShapeDtypeStruct((_EPAD, _H), jnp.float32)),
    )(g, p2, pt, w2s, w2p, w2o, b2s, b2p, b2o)


def _big_body(g_ref, pv_ref, wp_ref, w2s_ref, w2p_ref, w2o_ref,
              b2s_ref, b2p_ref, b2o_ref, ns_ref, npv_ref, no_ref):
    h = _relu(g_ref[...] + pv_ref[...] @ wp_ref[...])
    ns_ref[...] = _relu(h @ w2s_ref[...] + b2s_ref[...])
    npv_ref[...] = _relu(h @ w2p_ref[...] + b2p_ref[...])
    no_ref[...] = _relu(h @ w2o_ref[...] + b2o_ref[...])


def _big(g, pv, wp, w2s, w2p, w2o, b2s, b2p, b2o):
    eb = pl.BlockSpec((_EB, _H), lambda i: (i, 0))
    full = lambda shape: pl.BlockSpec(shape, lambda i: (0, 0))
    return pl.pallas_call(
        _big_body,
        grid=(_NEB,),
        in_specs=[eb, eb,
                  full((_H, _H)),
                  full((_H, _H)), full((_H, _H)), full((_H, _H)),
                  full((1, _H)), full((1, _H)), full((1, _H))],
        out_specs=(eb, eb, eb),
        out_shape=(jax.ShapeDtypeStruct((_EPAD, _H), jnp.float32),
                   jax.ShapeDtypeStruct((_EPAD, _H), jnp.float32),
                   jax.ShapeDtypeStruct((_EPAD, _H), jnp.float32)),
    )(g, pv, wp, w2s, w2p, w2o, b2s, b2p, b2o)


def _node_body(p0_ref, p1_ref, cnt_ref, w1_ref, nb1_ref, w2_ref, nb2_ref,
               wsn_ref, b1n_ref, won_ref, a_ref, b_ref):
    pooled = (p0_ref[...] + p1_ref[...]) / jnp.maximum(cnt_ref[...], 1.0)
    h2 = _relu(pooled @ w1_ref[...] + nb1_ref[...])
    ov = _relu(h2 @ w2_ref[...] + nb2_ref[...])
    a_ref[...] = ov @ wsn_ref[...] + b1n_ref[...]
    b_ref[...] = ov @ won_ref[...]


def _node(p0, p1, cnt, w1, nb1, w2, nb2, wsn, b1n, won):
    nb = pl.BlockSpec((_NB, _H), lambda i: (i, 0))
    full = lambda shape: pl.BlockSpec(shape, lambda i: (0, 0))
    return pl.pallas_call(
        _node_body,
        grid=(_NPAD // _NB,),
        in_specs=[nb, nb,
                  pl.BlockSpec((_NB, 1), lambda i: (i, 0)),
                  full((_H, _H)), full((1, _H)), full((_H, _H)), full((1, _H)),
                  full((_H, _H)), full((1, _H)), full((_H, _H))],
        out_specs=(nb, nb),
        out_shape=(jax.ShapeDtypeStruct((_NPAD, _H), jnp.float32),
                   jax.ShapeDtypeStruct((_NPAD, _H), jnp.float32)),
    )(p0, p1, cnt, w1, nb1, w2, nb2, wsn, b1n, won)


def _node_last_body(p0_ref, p1_ref, cnt_ref, w1_ref, nb1_ref, w2_ref, nb2_ref,
                    ov_ref):
    pooled = (p0_ref[...] + p1_ref[...]) / jnp.maximum(cnt_ref[...], 1.0)
    h2 = _relu(pooled @ w1_ref[...] + nb1_ref[...])
    ov_ref[...] = _relu(h2 @ w2_ref[...] + nb2_ref[...])


def _node_last(p0, p1, cnt, w1, nb1, w2, nb2):
    nb = pl.BlockSpec((_NB, _H), lambda i: (i, 0))
    full = lambda shape: pl.BlockSpec(shape, lambda i: (0, 0))
    return pl.pallas_call(
        _node_last_body,
        grid=(_NPAD // _NB,),
        in_specs=[nb, nb,
                  pl.BlockSpec((_NB, 1), lambda i: (i, 0)),
                  full((_H, _H)), full((1, _H)), full((_H, _H)), full((1, _H))],
        out_specs=nb,
        out_shape=jax.ShapeDtypeStruct((_NPAD, _H), jnp.float32),
    )(p0, p1, cnt, w1, nb1, w2, nb2)


_NC = 2
_NS = 16
_NW = _NC * _NS
_GSB = 256
_GBPW = _EPAD // _GSB // _NW
_EPW = _EPAD // _NW
_ROWS_PER_TILE = _NPAD // _NS

_sc_mesh = plsc.VectorSubcoreMesh(
    core_axis_name="c", subcore_axis_name="s", num_cores=_NC, num_subcores=_NS)


@functools.partial(
    pl.kernel,
    out_type=jax.ShapeDtypeStruct((_EPAD, _H), jnp.float32),
    mesh=_sc_mesh,
    scratch_types=[pltpu.VMEM((2 * _EPW,), jnp.int32),
                   pltpu.VMEM((_GSB, _H), jnp.float32),
                   pltpu.VMEM((_GSB, _H), jnp.float32),
                   pltpu.SemaphoreType.DMA,
                   pltpu.SemaphoreType.DMA],
)
def _gather_add(a_hbm, b_hbm, idx_hbm, g_hbm, idx_v, buf0, buf1, sem0, sem1):
    c = lax.axis_index("c")
    t = lax.axis_index("s")
    wid = c * _NS + t
    base = wid * _GBPW
    pltpu.sync_copy(idx_hbm.at[wid], idx_v)

    def body(i, carry):
        m0 = 2 * i
        blk0 = base + m0
        off0 = pl.multiple_of(m0 * (2 * _GSB), 2 * _GSB)
        off1 = off0 + 2 * _GSB
        ha0 = pltpu.async_copy(a_hbm.at[idx_v.at[pl.ds(off0, _GSB)]],
                               buf0, sem0)
        ha1 = pltpu.async_copy(a_hbm.at[idx_v.at[pl.ds(off1, _GSB)]],
                               buf1, sem1)
        ha0.wait()
        hb0 = pltpu.async_copy(b_hbm.at[idx_v.at[pl.ds(off0 + _GSB, _GSB)]],
                               buf0, sem0, add=True)
        ha1.wait()
        hb1 = pltpu.async_copy(b_hbm.at[idx_v.at[pl.ds(off1 + _GSB, _GSB)]],
                               buf1, sem1, add=True)
        hb0.wait()
        hw0 = pltpu.async_copy(buf0, g_hbm.at[pl.ds(blk0 * _GSB, _GSB)], sem0)
        hb1.wait()
        hw1 = pltpu.async_copy(buf1, g_hbm.at[pl.ds((blk0 + 1) * _GSB, _GSB)],
                               sem1)
        hw0.wait()
        hw1.wait()
        return carry

    lax.fori_loop(0, _GBPW // 2, body, 0)


@functools.partial(
    pl.kernel,
    out_type=jax.ShapeDtypeStruct((_NC, _NPAD, _H), jnp.float32),
    mesh=_sc_mesh,
    scratch_types=[pltpu.VMEM((128,), jnp.int32),
                   pltpu.VMEM((128,), jnp.int32),
                   pltpu.VMEM((128, _H), jnp.float32),
                   pltpu.VMEM((128, _H), jnp.float32),
                   pltpu.VMEM_SHARED((_NPAD, _H), jnp.float32),
                   pltpu.SemaphoreType.DMA,
                   pltpu.SemaphoreType.DMA],
)
def _scatter2(ns_hbm, no_hbm, idx_hbm, zrows_hbm, out_hbm,
              idx_s, idx_o, bufn, bufo, acc_sh, semn, semo):
    c = lax.axis_index("c")
    t = lax.axis_index("s")
    wid = c * _NS + t
    rbase = t * _ROWS_PER_TILE
    pltpu.sync_copy(zrows_hbm, acc_sh.at[pl.ds(rbase, _ROWS_PER_TILE)])
    plsc.subcore_barrier()
    nq = _EPW // 128

    def body(q, carry):
        e0 = (wid * nq + q) * 128
        off_s = (q // 2) * 512 + (q % 2) * 128
        hn = pltpu.async_copy(ns_hbm.at[pl.ds(e0, 128)], bufn, semn)
        ho = pltpu.async_copy(no_hbm.at[pl.ds(e0, 128)], bufo, semo)
        pltpu.sync_copy(idx_hbm.at[wid].at[pl.ds(off_s, 128)], idx_s)
        pltpu.sync_copy(idx_hbm.at[wid].at[pl.ds(off_s + 256, 128)], idx_o)
        hn.wait()
        hsn = pltpu.async_copy(bufn, acc_sh.at[idx_s], semn, add=True)
        ho.wait()
        hso = pltpu.async_copy(bufo, acc_sh.at[idx_o], semo, add=True)
        hsn.wait()
        hso.wait()
        return carry

    lax.fori_loop(0, nq, body, 0)
    plsc.subcore_barrier()
    pltpu.sync_copy(acc_sh.at[pl.ds(rbase, _ROWS_PER_TILE)],
                    out_hbm.at[c].at[pl.ds(rbase, _ROWS_PER_TILE)])


_CNT_RPT = 640
_CNT_PAD = _CNT_RPT * _NS


@functools.partial(
    pl.kernel,
    out_type=jax.ShapeDtypeStruct((_NC, _CNT_PAD), jnp.float32),
    mesh=_sc_mesh,
    scratch_types=[pltpu.VMEM((2 * _EPW,), jnp.int32),
                   pltpu.VMEM((2 * _EPW,), jnp.float32),
                   pltpu.VMEM_SHARED((_CNT_PAD,), jnp.float32),
                   pltpu.SemaphoreType.DMA],
)
def _counts(cidx_hbm, ones_hbm, z1_hbm, out_hbm, cidx_v, ones_v, cnt_sh, sem):
    c = lax.axis_index("c")
    t = lax.axis_index("s")
    wid = c * _NS + t
    rbase = t * _CNT_RPT
    pltpu.sync_copy(z1_hbm, cnt_sh.at[pl.ds(rbase, _CNT_RPT)])
    pltpu.sync_copy(cidx_hbm.at[wid], cidx_v)
    pltpu.sync_copy(ones_hbm, ones_v)
    plsc.subcore_barrier()
    pltpu.sync_copy(ones_v, cnt_sh.at[cidx_v], add=True)
    plsc.subcore_barrier()
    pltpu.sync_copy(cnt_sh.at[pl.ds(rbase, _CNT_RPT)],
                    out_hbm.at[c].at[pl.ds(rbase, _CNT_RPT)])


def kernel(objs, triples, obj_emb, pred_emb,
           n1w1, n1b1, n1w2, n1b2, n2w1, n2b1, n2w2, n2b2):
    objs = objs.astype(jnp.int32)
    triples = triples.astype(jnp.int32)
    s = triples[:, 0]
    p = triples[:, 1]
    o = triples[:, 2]
    pad = _EPAD - _E
    padn = 10000 + (jnp.arange(pad, dtype=jnp.int32) % 16)
    s_p = jnp.concatenate([s, padn])
    o_p = jnp.concatenate([o, padn])
    p_p = jnp.concatenate([p, jnp.full((pad,), _NPRED, jnp.int32)])
    objs2 = jnp.concatenate(
        [objs, jnp.full((_NPAD - _N,), 129, jnp.int32)]).reshape(_NPAD, 1)
    obj_emb_p = jnp.pad(obj_emb, ((0, _OBJ_PAD - 129), (0, 0)))

    Ws = [n1w1[i, :_H] for i in range(4)]
    Wp = [n1w1[i, _H:2 * _H] for i in range(4)]
    Wo = [n1w1[i, 2 * _H:] for i in range(4)]
    b1 = [n1b1[i].reshape(1, _H) for i in range(4)]
    W2s = [n1w2[i][:, :_H] for i in range(4)]
    W2p = [n1w2[i][:, _H:2 * _H] for i in range(4)]
    W2o = [n1w2[i][:, 2 * _H:] for i in range(4)]
    b2s = [n1b2[i][:_H].reshape(1, _H) for i in range(4)]
    b2p = [n1b2[i][_H:2 * _H].reshape(1, _H) for i in range(4)]
    b2o = [n1b2[i][2 * _H:].reshape(1, _H) for i in range(4)]
    W1n = [n2w1[i] for i in range(4)]
    b1n_ = [n2b1[i].reshape(1, _H) for i in range(4)]
    W2n = [n2w2[i] for i in range(4)]
    b2n_ = [n2b2[i].reshape(1, _H) for i in range(4)]

    idx_w = jnp.stack([s_p.reshape(_EPAD // _GSB, _GSB),
                       o_p.reshape(_EPAD // _GSB, _GSB)],
                      axis=1).reshape(_NW, 2 * _EPW)

    ts, to, pt = _pre(obj_emb_p, pred_emb, Ws[0], Wo[0], Wp[0], b1[0])
    A, B = _node0(objs2, ts, to)

    ones_w = jnp.ones((2 * _EPW,), jnp.float32)
    z1 = jnp.zeros((_CNT_RPT,), jnp.float32)
    zrows = jnp.zeros((_ROWS_PER_TILE, _H), jnp.float32)
    cnt_part = _counts(idx_w, ones_w, z1)
    cnt2 = (cnt_part[0, :_NPAD] + cnt_part[1, :_NPAD]).reshape(_NPAD, 1)

    pv = None
    ov = None
    npv = None
    for i in range(4):
        g = _gather_add(A, B, idx_w)
        if i == 0:
            ns, npv, no = _big0(g, p_p.reshape(_EPAD, 1), pt,
                                W2s[i], W2p[i], W2o[i], b2s[i], b2p[i], b2o[i])
        else:
            ns, npv, no = _big(g, pv, Wp[i],
                               W2s[i], W2p[i], W2o[i], b2s[i], b2p[i], b2o[i])
        pooled = _scatter2(ns, no, idx_w, zrows)
        if i < 3:
            A, B = _node(pooled[0], pooled[1], cnt2,
                         W1n[i], b1n_[i], W2n[i], b2n_[i],
                         Ws[i + 1], b1[i + 1], Wo[i + 1])
        else:
            ov = _node_last(pooled[0], pooled[1], cnt2,
                            W1n[i], b1n_[i], W2n[i], b2n_[i])
        pv = npv
    return ov[:_N], npv[:_E]


# speedup vs baseline: 5.1196x; 1.0382x over previous
import functools

import jax
import jax.numpy as jnp
from jax import lax
from jax.experimental import pallas as pl
from jax.experimental.pallas import tpu as pltpu
from jax.experimental.pallas import tpu_sc as plsc

_E = 320000
_EPAD = 327680
_EB = 2560
_NEB = _EPAD // _EB
_N = 10000
_NPAD = 10112
_NB = 2528
_OBJ_PAD = 136
_NPRED = 64
_H = 128


def _relu(x):
    return jnp.maximum(x, 0.0)


def _pre_body(obj_ref, pred_ref, ws_ref, wo_ref, wp_ref, b1_ref,
              ts_ref, to_ref, pt_ref):
    ts_ref[...] = obj_ref[...] @ ws_ref[...] + b1_ref[...]
    to_ref[...] = obj_ref[...] @ wo_ref[...]
    pt_ref[...] = pred_ref[...] @ wp_ref[...]


def _pre(obj_emb_p, pred_emb, ws, wo, wp, b1):
    return pl.pallas_call(
        _pre_body,
        out_shape=(jax.ShapeDtypeStruct((_OBJ_PAD, _H), jnp.float32),
                   jax.ShapeDtypeStruct((_OBJ_PAD, _H), jnp.float32),
                   jax.ShapeDtypeStruct((_NPRED, _H), jnp.float32)),
    )(obj_emb_p, pred_emb, ws, wo, wp, b1)


def _node0_body(objs_ref, ts_ref, to_ref, a_ref, b_ref):
    oh = (objs_ref[...] == lax.broadcasted_iota(jnp.int32, (_NB, _OBJ_PAD), 1)
          ).astype(jnp.float32)
    a_ref[...] = oh @ ts_ref[...]
    b_ref[...] = oh @ to_ref[...]


def _node0(objs2, ts, to):
    grid = (_NPAD // _NB,)
    return pl.pallas_call(
        _node0_body,
        grid=grid,
        in_specs=[pl.BlockSpec((_NB, 1), lambda i: (i, 0)),
                  pl.BlockSpec((_OBJ_PAD, _H), lambda i: (0, 0)),
                  pl.BlockSpec((_OBJ_PAD, _H), lambda i: (0, 0))],
        out_specs=(pl.BlockSpec((_NB, _H), lambda i: (i, 0)),
                   pl.BlockSpec((_NB, _H), lambda i: (i, 0))),
        out_shape=(jax.ShapeDtypeStruct((_NPAD, _H), jnp.float32),
                   jax.ShapeDtypeStruct((_NPAD, _H), jnp.float32)),
    )(objs2, ts, to)


def _big0_body(g_ref, p_ref, pt_ref, w2s_ref, w2p_ref, w2o_ref,
               b2s_ref, b2p_ref, b2o_ref, ns_ref, npv_ref, no_ref):
    oh = (p_ref[...] == lax.broadcasted_iota(jnp.int32, (_EB, _NPRED), 1)
          ).astype(jnp.float32)
    h = _relu(g_ref[...] + oh @ pt_ref[...])
    ns_ref[...] = _relu(h @ w2s_ref[...] + b2s_ref[...])
    npv_ref[...] = _relu(h @ w2p_ref[...] + b2p_ref[...]).astype(jnp.bfloat16)
    no_ref[...] = _relu(h @ w2o_ref[...] + b2o_ref[...])


def _big0(g, p2, pt, w2s, w2p, w2o, b2s, b2p, b2o):
    eb = pl.BlockSpec((_EB, _H), lambda i: (i, 0))
    full = lambda shape: pl.BlockSpec(shape, lambda i: (0, 0))
    return pl.pallas_call(
        _big0_body,
        grid=(_NEB,),
        in_specs=[eb,
                  pl.BlockSpec((_EB, 1), lambda i: (i, 0)),
                  full((_NPRED, _H)),
                  full((_H, _H)), full((_H, _H)), full((_H, _H)),
                  full((1, _H)), full((1, _H)), full((1, _H))],
        out_specs=(eb, eb, eb),
        out_shape=(jax.ShapeDtypeStruct((_EPAD, _H), jnp.float32),
                   jax.ShapeDtypeStruct((_EPAD, _H), jnp.bfloat16),
                   jax.ShapeDtypeStruct((_EPAD, _H), jnp.float32)),
    )(g, p2, pt, w2s, w2p, w2o, b2s, b2p, b2o)


def _big_body(g_ref, pv_ref, wp_ref, w2s_ref, w2p_ref, w2o_ref,
              b2s_ref, b2p_ref, b2o_ref, ns_ref, npv_ref, no_ref):
    h = _relu(g_ref[...] + pv_ref[...].astype(jnp.float32) @ wp_ref[...])
    ns_ref[...] = _relu(h @ w2s_ref[...] + b2s_ref[...])
    npv_ref[...] = _relu(h @ w2p_ref[...] + b2p_ref[...]).astype(jnp.bfloat16)
    no_ref[...] = _relu(h @ w2o_ref[...] + b2o_ref[...])


def _big(g, pv, wp, w2s, w2p, w2o, b2s, b2p, b2o):
    eb = pl.BlockSpec((_EB, _H), lambda i: (i, 0))
    full = lambda shape: pl.BlockSpec(shape, lambda i: (0, 0))
    return pl.pallas_call(
        _big_body,
        grid=(_NEB,),
        in_specs=[eb, eb,
                  full((_H, _H)),
                  full((_H, _H)), full((_H, _H)), full((_H, _H)),
                  full((1, _H)), full((1, _H)), full((1, _H))],
        out_specs=(eb, eb, eb),
        out_shape=(jax.ShapeDtypeStruct((_EPAD, _H), jnp.float32),
                   jax.ShapeDtypeStruct((_EPAD, _H), jnp.bfloat16),
                   jax.ShapeDtypeStruct((_EPAD, _H), jnp.float32)),
    )(g, pv, wp, w2s, w2p, w2o, b2s, b2p, b2o)


def _node_body(p0_ref, p1_ref, cnt_ref, w1_ref, nb1_ref, w2_ref, nb2_ref,
               wsn_ref, b1n_ref, won_ref, a_ref, b_ref):
    pooled = (p0_ref[...] + p1_ref[...]) / jnp.maximum(cnt_ref[...], 1.0)
    h2 = _relu(pooled @ w1_ref[...] + nb1_ref[...])
    ov = _relu(h2 @ w2_ref[...] + nb2_ref[...])
    a_ref[...] = ov @ wsn_ref[...] + b1n_ref[...]
    b_ref[...] = ov @ won_ref[...]


def _node(p0, p1, cnt, w1, nb1, w2, nb2, wsn, b1n, won):
    nb = pl.BlockSpec((_NB, _H), lambda i: (i, 0))
    full = lambda shape: pl.BlockSpec(shape, lambda i: (0, 0))
    return pl.pallas_call(
        _node_body,
        grid=(_NPAD // _NB,),
        in_specs=[nb, nb,
                  pl.BlockSpec((_NB, 1), lambda i: (i, 0)),
                  full((_H, _H)), full((1, _H)), full((_H, _H)), full((1, _H)),
                  full((_H, _H)), full((1, _H)), full((_H, _H))],
        out_specs=(nb, nb),
        out_shape=(jax.ShapeDtypeStruct((_NPAD, _H), jnp.float32),
                   jax.ShapeDtypeStruct((_NPAD, _H), jnp.float32)),
    )(p0, p1, cnt, w1, nb1, w2, nb2, wsn, b1n, won)


def _node_last_body(p0_ref, p1_ref, cnt_ref, w1_ref, nb1_ref, w2_ref, nb2_ref,
                    ov_ref):
    pooled = (p0_ref[...] + p1_ref[...]) / jnp.maximum(cnt_ref[...], 1.0)
    h2 = _relu(pooled @ w1_ref[...] + nb1_ref[...])
    ov_ref[...] = _relu(h2 @ w2_ref[...] + nb2_ref[...])


def _node_last(p0, p1, cnt, w1, nb1, w2, nb2):
    nb = pl.BlockSpec((_NB, _H), lambda i: (i, 0))
    full = lambda shape: pl.BlockSpec(shape, lambda i: (0, 0))
    return pl.pallas_call(
        _node_last_body,
        grid=(_NPAD // _NB,),
        in_specs=[nb, nb,
                  pl.BlockSpec((_NB, 1), lambda i: (i, 0)),
                  full((_H, _H)), full((1, _H)), full((_H, _H)), full((1, _H))],
        out_specs=nb,
        out_shape=jax.ShapeDtypeStruct((_NPAD, _H), jnp.float32),
    )(p0, p1, cnt, w1, nb1, w2, nb2)


_NC = 2
_NS = 16
_NW = _NC * _NS
_GSB = 256
_GBPW = _EPAD // _GSB // _NW
_EPW = _EPAD // _NW
_ROWS_PER_TILE = _NPAD // _NS

_sc_mesh = plsc.VectorSubcoreMesh(
    core_axis_name="c", subcore_axis_name="s", num_cores=_NC, num_subcores=_NS)


@functools.partial(
    pl.kernel,
    out_type=jax.ShapeDtypeStruct((_EPAD, _H), jnp.float32),
    mesh=_sc_mesh,
    scratch_types=[pltpu.VMEM((2 * _EPW,), jnp.int32),
                   pltpu.VMEM((_GSB, _H), jnp.float32),
                   pltpu.VMEM((_GSB, _H), jnp.float32),
                   pltpu.SemaphoreType.DMA,
                   pltpu.SemaphoreType.DMA],
)
def _gather_add(a_hbm, b_hbm, idx_hbm, g_hbm, idx_v, buf0, buf1, sem0, sem1):
    c = lax.axis_index("c")
    t = lax.axis_index("s")
    wid = c * _NS + t
    base = wid * _GBPW
    pltpu.sync_copy(idx_hbm.at[wid], idx_v)

    def body(i, carry):
        m0 = 2 * i
        blk0 = base + m0
        off0 = pl.multiple_of(m0 * (2 * _GSB), 2 * _GSB)
        off1 = off0 + 2 * _GSB
        ha0 = pltpu.async_copy(a_hbm.at[idx_v.at[pl.ds(off0, _GSB)]],
                               buf0, sem0)
        ha1 = pltpu.async_copy(a_hbm.at[idx_v.at[pl.ds(off1, _GSB)]],
                               buf1, sem1)
        ha0.wait()
        hb0 = pltpu.async_copy(b_hbm.at[idx_v.at[pl.ds(off0 + _GSB, _GSB)]],
                               buf0, sem0, add=True)
        ha1.wait()
        hb1 = pltpu.async_copy(b_hbm.at[idx_v.at[pl.ds(off1 + _GSB, _GSB)]],
                               buf1, sem1, add=True)
        hb0.wait()
        hw0 = pltpu.async_copy(buf0, g_hbm.at[pl.ds(blk0 * _GSB, _GSB)], sem0)
        hb1.wait()
        hw1 = pltpu.async_copy(buf1, g_hbm.at[pl.ds((blk0 + 1) * _GSB, _GSB)],
                               sem1)
        hw0.wait()
        hw1.wait()
        return carry

    lax.fori_loop(0, _GBPW // 2, body, 0)


@functools.partial(
    pl.kernel,
    out_type=jax.ShapeDtypeStruct((_NC, _NPAD, _H), jnp.float32),
    mesh=_sc_mesh,
    scratch_types=[pltpu.VMEM((128,), jnp.int32),
                   pltpu.VMEM((128,), jnp.int32),
                   pltpu.VMEM((128, _H), jnp.float32),
                   pltpu.VMEM((128, _H), jnp.float32),
                   pltpu.VMEM_SHARED((_NPAD, _H), jnp.float32),
                   pltpu.SemaphoreType.DMA,
                   pltpu.SemaphoreType.DMA],
)
def _scatter2(ns_hbm, no_hbm, idx_hbm, zrows_hbm, out_hbm,
              idx_s, idx_o, bufn, bufo, acc_sh, semn, semo):
    c = lax.axis_index("c")
    t = lax.axis_index("s")
    wid = c * _NS + t
    rbase = t * _ROWS_PER_TILE
    pltpu.sync_copy(zrows_hbm, acc_sh.at[pl.ds(rbase, _ROWS_PER_TILE)])
    plsc.subcore_barrier()
    nq = _EPW // 128

    def body(q, carry):
        e0 = (wid * nq + q) * 128
        off_s = (q // 2) * 512 + (q % 2) * 128
        hn = pltpu.async_copy(ns_hbm.at[pl.ds(e0, 128)], bufn, semn)
        ho = pltpu.async_copy(no_hbm.at[pl.ds(e0, 128)], bufo, semo)
        pltpu.sync_copy(idx_hbm.at[wid].at[pl.ds(off_s, 128)], idx_s)
        pltpu.sync_copy(idx_hbm.at[wid].at[pl.ds(off_s + 256, 128)], idx_o)
        hn.wait()
        hsn = pltpu.async_copy(bufn, acc_sh.at[idx_s], semn, add=True)
        ho.wait()
        hso = pltpu.async_copy(bufo, acc_sh.at[idx_o], semo, add=True)
        hsn.wait()
        hso.wait()
        return carry

    lax.fori_loop(0, nq, body, 0)
    plsc.subcore_barrier()
    pltpu.sync_copy(acc_sh.at[pl.ds(rbase, _ROWS_PER_TILE)],
                    out_hbm.at[c].at[pl.ds(rbase, _ROWS_PER_TILE)])


_CNT_RPT = 640
_CNT_PAD = _CNT_RPT * _NS


@functools.partial(
    pl.kernel,
    out_type=jax.ShapeDtypeStruct((_NC, _CNT_PAD), jnp.float32),
    mesh=_sc_mesh,
    scratch_types=[pltpu.VMEM((2 * _EPW,), jnp.int32),
                   pltpu.VMEM((2 * _EPW,), jnp.float32),
                   pltpu.VMEM_SHARED((_CNT_PAD,), jnp.float32),
                   pltpu.SemaphoreType.DMA],
)
def _counts(cidx_hbm, ones_hbm, z1_hbm, out_hbm, cidx_v, ones_v, cnt_sh, sem):
    c = lax.axis_index("c")
    t = lax.axis_index("s")
    wid = c * _NS + t
    rbase = t * _CNT_RPT
    pltpu.sync_copy(z1_hbm, cnt_sh.at[pl.ds(rbase, _CNT_RPT)])
    pltpu.sync_copy(cidx_hbm.at[wid], cidx_v)
    pltpu.sync_copy(ones_hbm, ones_v)
    plsc.subcore_barrier()
    pltpu.sync_copy(ones_v, cnt_sh.at[cidx_v], add=True)
    plsc.subcore_barrier()
    pltpu.sync_copy(cnt_sh.at[pl.ds(rbase, _CNT_RPT)],
                    out_hbm.at[c].at[pl.ds(rbase, _CNT_RPT)])


def kernel(objs, triples, obj_emb, pred_emb,
           n1w1, n1b1, n1w2, n1b2, n2w1, n2b1, n2w2, n2b2):
    objs = objs.astype(jnp.int32)
    triples = triples.astype(jnp.int32)
    s = triples[:, 0]
    p = triples[:, 1]
    o = triples[:, 2]
    pad = _EPAD - _E
    padn = 10000 + (jnp.arange(pad, dtype=jnp.int32) % 16)
    s_p = jnp.concatenate([s, padn])
    o_p = jnp.concatenate([o, padn])
    p_p = jnp.concatenate([p, jnp.full((pad,), _NPRED, jnp.int32)])
    objs2 = jnp.concatenate(
        [objs, jnp.full((_NPAD - _N,), 129, jnp.int32)]).reshape(_NPAD, 1)
    obj_emb_p = jnp.pad(obj_emb, ((0, _OBJ_PAD - 129), (0, 0)))

    Ws = [n1w1[i, :_H] for i in range(4)]
    Wp = [n1w1[i, _H:2 * _H] for i in range(4)]
    Wo = [n1w1[i, 2 * _H:] for i in range(4)]
    b1 = [n1b1[i].reshape(1, _H) for i in range(4)]
    W2s = [n1w2[i][:, :_H] for i in range(4)]
    W2p = [n1w2[i][:, _H:2 * _H] for i in range(4)]
    W2o = [n1w2[i][:, 2 * _H:] for i in range(4)]
    b2s = [n1b2[i][:_H].reshape(1, _H) for i in range(4)]
    b2p = [n1b2[i][_H:2 * _H].reshape(1, _H) for i in range(4)]
    b2o = [n1b2[i][2 * _H:].reshape(1, _H) for i in range(4)]
    W1n = [n2w1[i] for i in range(4)]
    b1n_ = [n2b1[i].reshape(1, _H) for i in range(4)]
    W2n = [n2w2[i] for i in range(4)]
    b2n_ = [n2b2[i].reshape(1, _H) for i in range(4)]

    idx_w = jnp.stack([s_p.reshape(_EPAD // _GSB, _GSB),
                       o_p.reshape(_EPAD // _GSB, _GSB)],
                      axis=1).reshape(_NW, 2 * _EPW)

    ts, to, pt = _pre(obj_emb_p, pred_emb, Ws[0], Wo[0], Wp[0], b1[0])
    A, B = _node0(objs2, ts, to)

    ones_w = jnp.ones((2 * _EPW,), jnp.float32)
    z1 = jnp.zeros((_CNT_RPT,), jnp.float32)
    zrows = jnp.zeros((_ROWS_PER_TILE, _H), jnp.float32)
    cnt_part = _counts(idx_w, ones_w, z1)
    cnt2 = (cnt_part[0, :_NPAD] + cnt_part[1, :_NPAD]).reshape(_NPAD, 1)

    pv = None
    ov = None
    npv = None
    for i in range(4):
        g = _gather_add(A, B, idx_w)
        if i == 0:
            ns, npv, no = _big0(g, p_p.reshape(_EPAD, 1), pt,
                                W2s[i], W2p[i], W2o[i], b2s[i], b2p[i], b2o[i])
        else:
            ns, npv, no = _big(g, pv, Wp[i],
                               W2s[i], W2p[i], W2o[i], b2s[i], b2p[i], b2o[i])
        pooled = _scatter2(ns, no, idx_w, zrows)
        if i < 3:
            A, B = _node(pooled[0], pooled[1], cnt2,
                         W1n[i], b1n_[i], W2n[i], b2n_[i],
                         Ws[i + 1], b1[i + 1], Wo[i + 1])
        else:
            ov = _node_last(pooled[0], pooled[1], cnt2,
                            W1n[i], b1n_[i], W2n[i], b2n_[i])
        pv = npv
    return ov[:_N], npv[:_E].astype(jnp.float32)


# speedup vs baseline: 5.8336x; 1.1395x over previous
import functools

import jax
import jax.numpy as jnp
from jax import lax
from jax.experimental import pallas as pl
from jax.experimental.pallas import tpu as pltpu
from jax.experimental.pallas import tpu_sc as plsc

_E = 320000
_EPAD = 327680
_ECH = _EPAD // 2
_EB = 2560
_NEB = _ECH // _EB
_N = 10000
_NPAD = 10112
_NB = 2528
_OBJ_PAD = 136
_NPRED = 64
_H = 128


def _relu(x):
    return jnp.maximum(x, 0.0)


def _pre_body(obj_ref, pred_ref, ws_ref, wo_ref, wp_ref, b1_ref,
              ts_ref, to_ref, pt_ref):
    ts_ref[...] = obj_ref[...] @ ws_ref[...] + b1_ref[...]
    to_ref[...] = obj_ref[...] @ wo_ref[...]
    pt_ref[...] = pred_ref[...] @ wp_ref[...]


def _pre(obj_emb_p, pred_emb, ws, wo, wp, b1):
    return pl.pallas_call(
        _pre_body,
        out_shape=(jax.ShapeDtypeStruct((_OBJ_PAD, _H), jnp.float32),
                   jax.ShapeDtypeStruct((_OBJ_PAD, _H), jnp.float32),
                   jax.ShapeDtypeStruct((_NPRED, _H), jnp.float32)),
    )(obj_emb_p, pred_emb, ws, wo, wp, b1)


def _node0_body(objs_ref, ts_ref, to_ref, a_ref, b_ref):
    oh = (objs_ref[...] == lax.broadcasted_iota(jnp.int32, (_NB, _OBJ_PAD), 1)
          ).astype(jnp.float32)
    a_ref[...] = oh @ ts_ref[...]
    b_ref[...] = oh @ to_ref[...]


def _node0(objs2, ts, to):
    grid = (_NPAD // _NB,)
    return pl.pallas_call(
        _node0_body,
        grid=grid,
        in_specs=[pl.BlockSpec((_NB, 1), lambda i: (i, 0)),
                  pl.BlockSpec((_OBJ_PAD, _H), lambda i: (0, 0)),
                  pl.BlockSpec((_OBJ_PAD, _H), lambda i: (0, 0))],
        out_specs=(pl.BlockSpec((_NB, _H), lambda i: (i, 0)),
                   pl.BlockSpec((_NB, _H), lambda i: (i, 0))),
        out_shape=(jax.ShapeDtypeStruct((_NPAD, _H), jnp.float32),
                   jax.ShapeDtypeStruct((_NPAD, _H), jnp.float32)),
    )(objs2, ts, to)


def _big0_body(g_ref, p_ref, pt_ref, w2s_ref, w2p_ref, w2o_ref,
               b2s_ref, b2p_ref, b2o_ref, ns_ref, npv_ref, no_ref):
    oh = (p_ref[...] == lax.broadcasted_iota(jnp.int32, (_EB, _NPRED), 1)
          ).astype(jnp.float32)
    h = _relu(g_ref[...] + oh @ pt_ref[...])
    ns_ref[...] = _relu(h @ w2s_ref[...] + b2s_ref[...])
    npv_ref[...] = _relu(h @ w2p_ref[...] + b2p_ref[...]).astype(jnp.bfloat16)
    no_ref[...] = _relu(h @ w2o_ref[...] + b2o_ref[...])


def _big0(g, p2, pt, w2s, w2p, w2o, b2s, b2p, b2o):
    eb = pl.BlockSpec((_EB, _H), lambda i: (i, 0))
    full = lambda shape: pl.BlockSpec(shape, lambda i: (0, 0))
    return pl.pallas_call(
        _big0_body,
        grid=(_NEB,),
        in_specs=[eb,
                  pl.BlockSpec((_EB, 1), lambda i: (i, 0)),
                  full((_NPRED, _H)),
                  full((_H, _H)), full((_H, _H)), full((_H, _H)),
                  full((1, _H)), full((1, _H)), full((1, _H))],
        out_specs=(eb, eb, eb),
        out_shape=(jax.ShapeDtypeStruct((_ECH, _H), jnp.float32),
                   jax.ShapeDtypeStruct((_ECH, _H), jnp.bfloat16),
                   jax.ShapeDtypeStruct((_ECH, _H), jnp.float32)),
    )(g, p2, pt, w2s, w2p, w2o, b2s, b2p, b2o)


def _big_body(g_ref, pv_ref, wp_ref, w2s_ref, w2p_ref, w2o_ref,
              b2s_ref, b2p_ref, b2o_ref, ns_ref, npv_ref, no_ref):
    h = _relu(g_ref[...] + pv_ref[...].astype(jnp.float32) @ wp_ref[...])
    ns_ref[...] = _relu(h @ w2s_ref[...] + b2s_ref[...])
    npv_ref[...] = _relu(h @ w2p_ref[...] + b2p_ref[...]).astype(jnp.bfloat16)
    no_ref[...] = _relu(h @ w2o_ref[...] + b2o_ref[...])


def _big(g, pv, wp, w2s, w2p, w2o, b2s, b2p, b2o):
    eb = pl.BlockSpec((_EB, _H), lambda i: (i, 0))
    full = lambda shape: pl.BlockSpec(shape, lambda i: (0, 0))
    return pl.pallas_call(
        _big_body,
        grid=(_NEB,),
        in_specs=[eb, eb,
                  full((_H, _H)),
                  full((_H, _H)), full((_H, _H)), full((_H, _H)),
                  full((1, _H)), full((1, _H)), full((1, _H))],
        out_specs=(eb, eb, eb),
        out_shape=(jax.ShapeDtypeStruct((_ECH, _H), jnp.float32),
                   jax.ShapeDtypeStruct((_ECH, _H), jnp.bfloat16),
                   jax.ShapeDtypeStruct((_ECH, _H), jnp.float32)),
    )(g, pv, wp, w2s, w2p, w2o, b2s, b2p, b2o)


def _node_body(p0_ref, p1_ref, p2_ref, p3_ref, cnt_ref,
               w1_ref, nb1_ref, w2_ref, nb2_ref,
               wsn_ref, b1n_ref, won_ref, a_ref, b_ref):
    pooled = ((p0_ref[...] + p1_ref[...]) + (p2_ref[...] + p3_ref[...])
              ) / jnp.maximum(cnt_ref[...], 1.0)
    h2 = _relu(pooled @ w1_ref[...] + nb1_ref[...])
    ov = _relu(h2 @ w2_ref[...] + nb2_ref[...])
    a_ref[...] = ov @ wsn_ref[...] + b1n_ref[...]
    b_ref[...] = ov @ won_ref[...]


def _node(p4, cnt, w1, nb1, w2, nb2, wsn, b1n, won):
    nb = pl.BlockSpec((_NB, _H), lambda i: (i, 0))
    full = lambda shape: pl.BlockSpec(shape, lambda i: (0, 0))
    return pl.pallas_call(
        _node_body,
        grid=(_NPAD // _NB,),
        in_specs=[nb, nb, nb, nb,
                  pl.BlockSpec((_NB, 1), lambda i: (i, 0)),
                  full((_H, _H)), full((1, _H)), full((_H, _H)), full((1, _H)),
                  full((_H, _H)), full((1, _H)), full((_H, _H))],
        out_specs=(nb, nb),
        out_shape=(jax.ShapeDtypeStruct((_NPAD, _H), jnp.float32),
                   jax.ShapeDtypeStruct((_NPAD, _H), jnp.float32)),
    )(*p4, cnt, w1, nb1, w2, nb2, wsn, b1n, won)


def _node_last_body(p0_ref, p1_ref, p2_ref, p3_ref, cnt_ref,
                    w1_ref, nb1_ref, w2_ref, nb2_ref, ov_ref):
    pooled = ((p0_ref[...] + p1_ref[...]) + (p2_ref[...] + p3_ref[...])
              ) / jnp.maximum(cnt_ref[...], 1.0)
    h2 = _relu(pooled @ w1_ref[...] + nb1_ref[...])
    ov_ref[...] = _relu(h2 @ w2_ref[...] + nb2_ref[...])


def _node_last(p4, cnt, w1, nb1, w2, nb2):
    nb = pl.BlockSpec((_NB, _H), lambda i: (i, 0))
    full = lambda shape: pl.BlockSpec(shape, lambda i: (0, 0))
    return pl.pallas_call(
        _node_last_body,
        grid=(_NPAD // _NB,),
        in_specs=[nb, nb, nb, nb,
                  pl.BlockSpec((_NB, 1), lambda i: (i, 0)),
                  full((_H, _H)), full((1, _H)), full((_H, _H)), full((1, _H))],
        out_specs=nb,
        out_shape=jax.ShapeDtypeStruct((_NPAD, _H), jnp.float32),
    )(*p4, cnt, w1, nb1, w2, nb2)


_NC = 2
_NS = 16
_NW = _NC * _NS
_GSB = 256
_GBPW = _ECH // _GSB // _NW
_EPW = _EPAD // _NW
_EPWC = _ECH // _NW
_ROWS_PER_TILE = _NPAD // _NS

_sc_mesh = plsc.VectorSubcoreMesh(
    core_axis_name="c", subcore_axis_name="s", num_cores=_NC, num_subcores=_NS)


@functools.partial(
    pl.kernel,
    out_type=jax.ShapeDtypeStruct((_ECH, _H), jnp.float32),
    mesh=_sc_mesh,
    scratch_types=[pltpu.VMEM((2 * _EPWC,), jnp.int32),
                   pltpu.VMEM((_GSB, _H), jnp.float32),
                   pltpu.VMEM((_GSB, _H), jnp.float32),
                   pltpu.SemaphoreType.DMA,
                   pltpu.SemaphoreType.DMA],
)
def _gather_add(a_hbm, b_hbm, idx_hbm, g_hbm, idx_v, buf0, buf1, sem0, sem1):
    c = lax.axis_index("c")
    t = lax.axis_index("s")
    wid = c * _NS + t
    base = wid * _GBPW
    pltpu.sync_copy(idx_hbm.at[wid], idx_v)

    def body(i, carry):
        m0 = 2 * i
        blk0 = base + m0
        off0 = pl.multiple_of(m0 * (2 * _GSB), 2 * _GSB)
        off1 = off0 + 2 * _GSB
        ha0 = pltpu.async_copy(a_hbm.at[idx_v.at[pl.ds(off0, _GSB)]],
                               buf0, sem0)
        ha1 = pltpu.async_copy(a_hbm.at[idx_v.at[pl.ds(off1, _GSB)]],
                               buf1, sem1)
        ha0.wait()
        hb0 = pltpu.async_copy(b_hbm.at[idx_v.at[pl.ds(off0 + _GSB, _GSB)]],
                               buf0, sem0, add=True)
        ha1.wait()
        hb1 = pltpu.async_copy(b_hbm.at[idx_v.at[pl.ds(off1 + _GSB, _GSB)]],
                               buf1, sem1, add=True)
        hb0.wait()
        hw0 = pltpu.async_copy(buf0, g_hbm.at[pl.ds(blk0 * _GSB, _GSB)], sem0)
        hb1.wait()
        hw1 = pltpu.async_copy(buf1, g_hbm.at[pl.ds((blk0 + 1) * _GSB, _GSB)],
                               sem1)
        hw0.wait()
        hw1.wait()
        return carry

    lax.fori_loop(0, _GBPW // 2, body, 0)


@functools.partial(
    pl.kernel,
    out_type=jax.ShapeDtypeStruct((_NC, _NPAD, _H), jnp.float32),
    mesh=_sc_mesh,
    scratch_types=[pltpu.VMEM((128,), jnp.int32),
                   pltpu.VMEM((128,), jnp.int32),
                   pltpu.VMEM((128, _H), jnp.float32),
                   pltpu.VMEM((128, _H), jnp.float32),
                   pltpu.VMEM_SHARED((_NPAD, _H), jnp.float32),
                   pltpu.SemaphoreType.DMA,
                   pltpu.SemaphoreType.DMA],
)
def _scatter2(ns_hbm, no_hbm, idx_hbm, zrows_hbm, out_hbm,
              idx_s, idx_o, bufn, bufo, acc_sh, semn, semo):
    c = lax.axis_index("c")
    t = lax.axis_index("s")
    wid = c * _NS + t
    rbase = t * _ROWS_PER_TILE
    pltpu.sync_copy(zrows_hbm, acc_sh.at[pl.ds(rbase, _ROWS_PER_TILE)])
    plsc.subcore_barrier()
    nq = _EPWC // 128

    def body(q, carry):
        e0 = (wid * nq + q) * 128
        off_s = (q // 2) * 512 + (q % 2) * 128
        hn = pltpu.async_copy(ns_hbm.at[pl.ds(e0, 128)], bufn, semn)
        ho = pltpu.async_copy(no_hbm.at[pl.ds(e0, 128)], bufo, semo)
        pltpu.sync_copy(idx_hbm.at[wid].at[pl.ds(off_s, 128)], idx_s)
        pltpu.sync_copy(idx_hbm.at[wid].at[pl.ds(off_s + 256, 128)], idx_o)
        hn.wait()
        hsn = pltpu.async_copy(bufn, acc_sh.at[idx_s], semn, add=True)
        ho.wait()
        hso = pltpu.async_copy(bufo, acc_sh.at[idx_o], semo, add=True)
        hsn.wait()
        hso.wait()
        return carry

    lax.fori_loop(0, nq, body, 0)
    plsc.subcore_barrier()
    pltpu.sync_copy(acc_sh.at[pl.ds(rbase, _ROWS_PER_TILE)],
                    out_hbm.at[c].at[pl.ds(rbase, _ROWS_PER_TILE)])


_CNT_RPT = 640
_CNT_PAD = _CNT_RPT * _NS


@functools.partial(
    pl.kernel,
    out_type=jax.ShapeDtypeStruct((_NC, _CNT_PAD), jnp.float32),
    mesh=_sc_mesh,
    scratch_types=[pltpu.VMEM((2 * _EPW,), jnp.int32),
                   pltpu.VMEM((2 * _EPW,), jnp.float32),
                   pltpu.VMEM_SHARED((_CNT_PAD,), jnp.float32),
                   pltpu.SemaphoreType.DMA],
)
def _counts(cidx_hbm, ones_hbm, z1_hbm, out_hbm, cidx_v, ones_v, cnt_sh, sem):
    c = lax.axis_index("c")
    t = lax.axis_index("s")
    wid = c * _NS + t
    rbase = t * _CNT_RPT
    pltpu.sync_copy(z1_hbm, cnt_sh.at[pl.ds(rbase, _CNT_RPT)])
    pltpu.sync_copy(cidx_hbm.at[wid], cidx_v)
    pltpu.sync_copy(ones_hbm, ones_v)
    plsc.subcore_barrier()
    pltpu.sync_copy(ones_v, cnt_sh.at[cidx_v], add=True)
    plsc.subcore_barrier()
    pltpu.sync_copy(cnt_sh.at[pl.ds(rbase, _CNT_RPT)],
                    out_hbm.at[c].at[pl.ds(rbase, _CNT_RPT)])


def kernel(objs, triples, obj_emb, pred_emb,
           n1w1, n1b1, n1w2, n1b2, n2w1, n2b1, n2w2, n2b2):
    objs = objs.astype(jnp.int32)
    triples = triples.astype(jnp.int32)
    s = triples[:, 0]
    p = triples[:, 1]
    o = triples[:, 2]
    pad = _EPAD - _E
    padn = 10000 + (jnp.arange(pad, dtype=jnp.int32) % 16)
    s_p = jnp.concatenate([s, padn])
    o_p = jnp.concatenate([o, padn])
    p_p = jnp.concatenate([p, jnp.full((pad,), _NPRED, jnp.int32)])
    objs2 = jnp.concatenate(
        [objs, jnp.full((_NPAD - _N,), 129, jnp.int32)]).reshape(_NPAD, 1)
    obj_emb_p = jnp.pad(obj_emb, ((0, _OBJ_PAD - 129), (0, 0)))

    Ws = [n1w1[i, :_H] for i in range(4)]
    Wp = [n1w1[i, _H:2 * _H] for i in range(4)]
    Wo = [n1w1[i, 2 * _H:] for i in range(4)]
    b1 = [n1b1[i].reshape(1, _H) for i in range(4)]
    W2s = [n1w2[i][:, :_H] for i in range(4)]
    W2p = [n1w2[i][:, _H:2 * _H] for i in range(4)]
    W2o = [n1w2[i][:, 2 * _H:] for i in range(4)]
    b2s = [n1b2[i][:_H].reshape(1, _H) for i in range(4)]
    b2p = [n1b2[i][_H:2 * _H].reshape(1, _H) for i in range(4)]
    b2o = [n1b2[i][2 * _H:].reshape(1, _H) for i in range(4)]
    W1n = [n2w1[i] for i in range(4)]
    b1n_ = [n2b1[i].reshape(1, _H) for i in range(4)]
    W2n = [n2w2[i] for i in range(4)]
    b2n_ = [n2b2[i].reshape(1, _H) for i in range(4)]

    idx_w = jnp.stack([s_p.reshape(_EPAD // _GSB, _GSB),
                       o_p.reshape(_EPAD // _GSB, _GSB)],
                      axis=1).reshape(_NW, 2 * _EPW)
    idx_c = []
    for c in range(2):
        sl = slice(c * _ECH, (c + 1) * _ECH)
        idx_c.append(jnp.stack([s_p[sl].reshape(_ECH // _GSB, _GSB),
                                o_p[sl].reshape(_ECH // _GSB, _GSB)],
                               axis=1).reshape(_NW, 2 * _EPWC))

    ts, to, pt = _pre(obj_emb_p, pred_emb, Ws[0], Wo[0], Wp[0], b1[0])
    A, B = _node0(objs2, ts, to)

    ones_w = jnp.ones((2 * _EPW,), jnp.float32)
    z1 = jnp.zeros((_CNT_RPT,), jnp.float32)
    zrows = jnp.zeros((_ROWS_PER_TILE, _H), jnp.float32)
    cnt_part = _counts(idx_w, ones_w, z1)
    cnt2 = (cnt_part[0, :_NPAD] + cnt_part[1, :_NPAD]).reshape(_NPAD, 1)

    pv = [None, None]
    ov = None
    npv = [None, None]
    for i in range(4):
        p4 = []
        for c in range(2):
            g = _gather_add(A, B, idx_c[c])
            if i == 0:
                pc = p_p[c * _ECH:(c + 1) * _ECH].reshape(_ECH, 1)
                ns, npv[c], no = _big0(g, pc, pt, W2s[i], W2p[i], W2o[i],
                                       b2s[i], b2p[i], b2o[i])
            else:
                ns, npv[c], no = _big(g, pv[c], Wp[i], W2s[i], W2p[i], W2o[i],
                                      b2s[i], b2p[i], b2o[i])
            pooled = _scatter2(ns, no, idx_c[c], zrows)
            p4.extend([pooled[0], pooled[1]])
            pv[c] = npv[c]
        if i < 3:
            A, B = _node(p4, cnt2, W1n[i], b1n_[i], W2n[i], b2n_[i],
                         Ws[i + 1], b1[i + 1], Wo[i + 1])
        else:
            ov = _node_last(p4, cnt2, W1n[i], b1n_[i], W2n[i], b2n_[i])
    pred = jnp.concatenate(npv, axis=0)
    return ov[:_N], pred[:_E].astype(jnp.float32)
